# baseline (speedup 1.0000x reference)
import functools

import jax
import jax.numpy as jnp
import numpy as np
from jax import lax
from jax.experimental import pallas as pl
from jax.experimental.pallas import tpu as pltpu
from jax.experimental.pallas import tpu_sc as plsc

VOCAB = 1000000
EMBED_DIM = 32
SEQ = 50
BATCH = 4096
NC, NS, LANES = 2, 16, 16
NW = NC * NS
SLABS = 2 * BATCH
SLABS_PER_CHUNK = 2
TOK_PER_CHUNK = SLABS_PER_CHUNK * SEQ
CHUNK_FLOATS = TOK_PER_CHUNK * EMBED_DIM
CHUNKS_PER_W = SLABS // (NW * SLABS_PER_CHUNK)


def _make_gather_idx() -> np.ndarray:
    o = np.arange(2 * TOK_PER_CHUNK * LANES, dtype=np.int32)
    s, r = np.divmod(o, EMBED_DIM * SEQ)
    d, l = np.divmod(r, SEQ)
    trow = (s * SEQ + l).reshape(2 * TOK_PER_CHUNK, LANES)
    tcol = d.reshape(2 * TOK_PER_CHUNK, LANES)
    return np.stack([trow, tcol]).astype(np.int32)


_TIDX = _make_gather_idx()


def _body(table_hbm, idx_hbm, tidx_hbm, out_hbm, idx_v, tidx_v, rows_v,
          obuf_v, gsem, wsem):
    c = lax.axis_index("c")
    s = lax.axis_index("s")
    w = s * NC + c
    pltpu.sync_copy(idx_hbm.at[w], idx_v)
    pltpu.sync_copy(tidx_hbm, tidx_v)

    @pl.loop(0, CHUNKS_PER_W)
    def chunk(g):
        pltpu.async_copy(table_hbm.at[idx_v.at[g]], rows_v, gsem).wait()
        for m in range(2 * TOK_PER_CHUNK):
            v = plsc.load_gather(rows_v, [tidx_v[0, m], tidx_v[1, m]])
            obuf_v[pl.ds(m * LANES, LANES)] = v
        pltpu.async_copy(obuf_v, out_hbm.at[w, g], wsem).wait()


@functools.partial(jax.jit, donate_argnums=())
def _run(table, idx3, tidx):
    mesh = plsc.VectorSubcoreMesh(core_axis_name="c", subcore_axis_name="s",
                                  num_cores=NC, num_subcores=NS)
    kern = pl.kernel(
        _body,
        out_type=jax.ShapeDtypeStruct((NW, CHUNKS_PER_W, CHUNK_FLOATS),
                                      jnp.float32),
        mesh=mesh,
        scratch_types=[
            pltpu.VMEM((CHUNKS_PER_W, TOK_PER_CHUNK), jnp.int32),
            pltpu.VMEM((2, 2 * TOK_PER_CHUNK, LANES), jnp.int32),
            pltpu.VMEM((TOK_PER_CHUNK, EMBED_DIM), jnp.float32),
            pltpu.VMEM((CHUNK_FLOATS,), jnp.float32),
            pltpu.SemaphoreType.DMA,
            pltpu.SemaphoreType.DMA,
        ],
        compiler_params=pltpu.CompilerParams(needs_layout_passes=False,
                                             use_tc_tiling_on_sc=False),
    )
    return kern(table, idx3, tidx)


def kernel(query_input, document_input, table):
    idx = jnp.stack([query_input, document_input]).astype(jnp.int32)
    idx3 = idx.reshape(NW, CHUNKS_PER_W, TOK_PER_CHUNK)
    tidx = jnp.asarray(_TIDX)
    out = _run(table, idx3, tidx)
    return out.reshape(2, BATCH, EMBED_DIM, SEQ)


# speedup vs baseline: 1.0994x; 1.0994x over previous
import functools

import jax
import jax.numpy as jnp
import numpy as np
from jax import lax
from jax.experimental import pallas as pl
from jax.experimental.pallas import tpu as pltpu
from jax.experimental.pallas import tpu_sc as plsc

VOCAB = 1000000
EMBED_DIM = 32
SEQ = 50
BATCH = 4096
NC, NS, LANES = 2, 16, 16
NW = NC * NS
SLABS = 2 * BATCH
SLABS_PER_CHUNK = 2
TOK_PER_CHUNK = SLABS_PER_CHUNK * SEQ
CHUNK_FLOATS = TOK_PER_CHUNK * EMBED_DIM
CHUNKS_PER_W = SLABS // (NW * SLABS_PER_CHUNK)
NBUF = 2


def _make_gather_idx() -> np.ndarray:
    o = np.arange(2 * TOK_PER_CHUNK * LANES, dtype=np.int32)
    s, r = np.divmod(o, EMBED_DIM * SEQ)
    d, l = np.divmod(r, SEQ)
    trow = (s * SEQ + l).reshape(2 * TOK_PER_CHUNK, LANES)
    tcol = d.reshape(2 * TOK_PER_CHUNK, LANES)
    return np.stack([trow, tcol]).astype(np.int32)


_TIDX = _make_gather_idx()


def _body(table_hbm, idx_hbm, tidx_hbm, out_hbm, idx_v, tidx_v, rows_v,
          obuf_v, gsems, wsems):
    c = lax.axis_index("c")
    s = lax.axis_index("s")
    w = s * NC + c
    pltpu.sync_copy(idx_hbm.at[w], idx_v)
    pltpu.sync_copy(tidx_hbm, tidx_v)

    nbuf = len(gsems)
    for b in range(nbuf):
        pltpu.async_copy(table_hbm.at[idx_v.at[b]], rows_v.at[b], gsems[b])

    @pl.loop(0, CHUNKS_PER_W // nbuf)
    def chunk(g):
        for b in range(nbuf):
            cg = g * nbuf + b
            pltpu.make_async_copy(table_hbm.at[idx_v.at[cg]], rows_v.at[b],
                                  gsems[b]).wait()

            @pl.when(g >= 1)
            def _():
                pltpu.make_async_copy(obuf_v.at[b], out_hbm.at[w, cg - nbuf],
                                      wsems[b]).wait()

            for m in range(2 * TOK_PER_CHUNK):
                v = plsc.load_gather(rows_v.at[b],
                                     [tidx_v[0, m], tidx_v[1, m]])
                obuf_v[b, pl.ds(m * LANES, LANES)] = v
            pltpu.async_copy(obuf_v.at[b], out_hbm.at[w, cg], wsems[b])

            @pl.when(cg + nbuf < CHUNKS_PER_W)
            def _():
                pltpu.async_copy(table_hbm.at[idx_v.at[cg + nbuf]],
                                 rows_v.at[b], gsems[b])

    for b in range(nbuf):
        pltpu.make_async_copy(obuf_v.at[b],
                              out_hbm.at[w, CHUNKS_PER_W - nbuf + b],
                              wsems[b]).wait()


@functools.partial(jax.jit, donate_argnums=())
def _run(table, idx3, tidx):
    mesh = plsc.VectorSubcoreMesh(core_axis_name="c", subcore_axis_name="s",
                                  num_cores=NC, num_subcores=NS)
    kern = pl.kernel(
        _body,
        out_type=jax.ShapeDtypeStruct((NW, CHUNKS_PER_W, CHUNK_FLOATS),
                                      jnp.float32),
        mesh=mesh,
        scratch_types=[
            pltpu.VMEM((CHUNKS_PER_W, TOK_PER_CHUNK), jnp.int32),
            pltpu.VMEM((2, 2 * TOK_PER_CHUNK, LANES), jnp.int32),
            pltpu.VMEM((NBUF, TOK_PER_CHUNK, EMBED_DIM), jnp.float32),
            pltpu.VMEM((NBUF, CHUNK_FLOATS), jnp.float32),
            [pltpu.SemaphoreType.DMA] * NBUF,
            [pltpu.SemaphoreType.DMA] * NBUF,
        ],
        compiler_params=pltpu.CompilerParams(needs_layout_passes=False,
                                             use_tc_tiling_on_sc=False),
    )
    return kern(table, idx3, tidx)


def kernel(query_input, document_input, table):
    idx = jnp.stack([query_input, document_input]).astype(jnp.int32)
    idx3 = idx.reshape(NW, CHUNKS_PER_W, TOK_PER_CHUNK)
    tidx = jnp.asarray(_TIDX)
    out = _run(table, idx3, tidx)
    return out.reshape(2, BATCH, EMBED_DIM, SEQ)
